# baseline (speedup 1.0000x reference)
import jax
import jax.numpy as jnp
from jax import lax
from jax.experimental import pallas as pl
from jax.experimental.pallas import tpu as pltpu, tpu_sc as plsc

N_NODES = 10000
N_EDGES = 320000
D = 128

NC = 2
NS = 16
NW = NC * NS
NCH = 80
TCH = NW * NCH
E_PAD = TCH * 128
DUMMY = N_NODES
ACC = 10240
RPT = ACC // NS

QB = 32
C0 = 160
C1 = 0
NB0 = C0 // QB
NB1 = C1 // QB

_MESH = plsc.VectorSubcoreMesh(core_axis_name="c", subcore_axis_name="s")


def _agg_body(y_hbm, src_hbm, dst_hbm, p_out, src_v, dst_v, r0, r1,
              acc_s, s0, s1):
    cid = lax.axis_index("c")
    sid = lax.axis_index("s")
    wid = cid * NS + sid
    base = sid * RPT

    zero16 = jnp.zeros((16,), jnp.float32)

    def zrow(i, _):
        def zcol(j, _):
            r0[i, pl.ds(j * 16, 16)] = zero16
            return 0
        return lax.fori_loop(0, D // 16, zcol, 0)
    lax.fori_loop(0, 128, zrow, 0)

    def zcp(k, _):
        pltpu.sync_copy(r0, acc_s.at[pl.ds(base + k * 128, 128)])
        return 0
    lax.fori_loop(0, RPT // 128, zcp, 0)

    plsc.subcore_barrier()

    def gather(j, r, s):
        pltpu.async_copy(y_hbm.at[src_v.at[j]], r, s)

    def gwait(r, s):
        pltpu.make_async_copy(y_hbm.at[pl.ds(0, 128)], r, s).wait()

    start = jnp.where(cid == 0, sid * C0, NS * C0 + sid * C1)
    nb = jnp.where(cid == 0, NB0, NB1)

    def block(b, _):
        c0 = start + b * QB
        pltpu.sync_copy(src_hbm.at[pl.ds(c0, QB)], src_v)
        pltpu.sync_copy(dst_hbm.at[pl.ds(c0, QB)], dst_v)

        gather(0, r0, s0)
        gather(1, r1, s1)

        def pair(k, _):
            gwait(r0, s0)
            pltpu.sync_copy(r0, acc_s.at[dst_v.at[2 * k]], add=True)
            gather(2 * k + 2, r0, s0)
            gwait(r1, s1)
            pltpu.sync_copy(r1, acc_s.at[dst_v.at[2 * k + 1]], add=True)
            gather(2 * k + 3, r1, s1)
            return 0
        lax.fori_loop(0, QB // 2 - 1, pair, 0)

        gwait(r0, s0)
        pltpu.sync_copy(r0, acc_s.at[dst_v.at[QB - 2]], add=True)
        gwait(r1, s1)
        pltpu.sync_copy(r1, acc_s.at[dst_v.at[QB - 1]], add=True)
        return 0

    lax.fori_loop(0, nb, block, 0)

    plsc.subcore_barrier()

    off = cid * ACC + base
    pltpu.sync_copy(acc_s.at[pl.ds(base, RPT)], p_out.at[pl.ds(off, RPT)])


_sc_agg = pl.kernel(
    _agg_body,
    out_type=jax.ShapeDtypeStruct((NC * ACC, D), jnp.float32),
    mesh=_MESH,
    scratch_types=[
        pltpu.VMEM((QB, 128), jnp.int32),
        pltpu.VMEM((QB, 128), jnp.int32),
        pltpu.VMEM((128, D), jnp.float32),
        pltpu.VMEM((128, D), jnp.float32),
        pltpu.VMEM_SHARED((ACC, D), jnp.float32),
        pltpu.SemaphoreType.DMA,
        pltpu.SemaphoreType.DMA,
    ],
)


def _deg_body(dst_hbm, d_out, dst_v, ones_v, zbuf, dacc_s):
    cid = lax.axis_index("c")
    sid = lax.axis_index("s")
    wid = cid * NS + sid
    base = sid * RPT

    zero16 = jnp.zeros((16,), jnp.float32)
    one16 = jnp.ones((16,), jnp.float32)

    def zrow(i, _):
        def zcol(j, _):
            zbuf[i, pl.ds(j * 16, 16)] = zero16
            return 0
        return lax.fori_loop(0, D // 16, zcol, 0)
    lax.fori_loop(0, 16, zrow, 0)

    def orow(i, _):
        def ocol(j, _):
            ones_v[i, pl.ds(j * 16, 16)] = one16
            return 0
        return lax.fori_loop(0, D // 16, ocol, 0)
    lax.fori_loop(0, 128, orow, 0)

    def zcd(k, _):
        pltpu.sync_copy(zbuf, dacc_s.at[pl.ds(base + k * 16, 16)])
        return 0
    lax.fori_loop(0, RPT // 16, zcd, 0)

    plsc.subcore_barrier()

    def quarter(q):
        pltpu.sync_copy(dst_hbm.at[pl.ds(wid * NCH + q * 16, 16)], dst_v)

        def step(j, _):
            pltpu.sync_copy(ones_v, dacc_s.at[dst_v.at[j]], add=True)
            return 0
        lax.fori_loop(0, 16, step, 0)

    for q in range(5):
        quarter(q)

    plsc.subcore_barrier()

    off = cid * ACC + base
    pltpu.sync_copy(dacc_s.at[pl.ds(base, RPT)], d_out.at[pl.ds(off, RPT)])


_sc_deg = pl.kernel(
    _deg_body,
    out_type=jax.ShapeDtypeStruct((NC * ACC, D), jnp.float32),
    mesh=_MESH,
    scratch_types=[
        pltpu.VMEM((16, 128), jnp.int32),
        pltpu.VMEM((128, D), jnp.float32),
        pltpu.VMEM((16, D), jnp.float32),
        pltpu.VMEM_SHARED((ACC, D), jnp.float32),
    ],
)


BS = 2000


def _mm_body(x_ref, w_ref, o_ref):
    o_ref[...] = jnp.dot(x_ref[...], w_ref[...], preferred_element_type=jnp.float32)


_mm = pl.pallas_call(
    _mm_body,
    grid=(N_NODES // BS,),
    in_specs=[
        pl.BlockSpec((BS, D), lambda i: (i, 0)),
        pl.BlockSpec((D, D), lambda i: (0, 0)),
    ],
    out_specs=pl.BlockSpec((BS, D), lambda i: (i, 0)),
    out_shape=jax.ShapeDtypeStruct((N_NODES, D), jnp.float32),
)


def _fuse1_body(x_ref, ws_ref, b_ref, p_ref, d_ref, wn2_ref, h_ref, y2_ref):
    psum = p_ref[0] + p_ref[1]
    dsum = d_ref[0] + d_ref[1]
    rdeg = 1.0 / jnp.maximum(dsum, 1.0)
    h = (jnp.dot(x_ref[...], ws_ref[...], preferred_element_type=jnp.float32)
         + psum * rdeg + b_ref[...])
    h = jnp.maximum(h, 0.0)
    h_ref[...] = h
    y2_ref[...] = jnp.dot(h, wn2_ref[...], preferred_element_type=jnp.float32)


_fuse1 = pl.pallas_call(
    _fuse1_body,
    grid=(N_NODES // BS,),
    in_specs=[
        pl.BlockSpec((BS, D), lambda i: (i, 0)),
        pl.BlockSpec((D, D), lambda i: (0, 0)),
        pl.BlockSpec((1, D), lambda i: (0, 0)),
        pl.BlockSpec((NC, BS, D), lambda i: (0, i, 0)),
        pl.BlockSpec((NC, BS, 1), lambda i: (0, i, 0)),
        pl.BlockSpec((D, D), lambda i: (0, 0)),
    ],
    out_specs=[
        pl.BlockSpec((BS, D), lambda i: (i, 0)),
        pl.BlockSpec((BS, D), lambda i: (i, 0)),
    ],
    out_shape=[
        jax.ShapeDtypeStruct((N_NODES, D), jnp.float32),
        jax.ShapeDtypeStruct((N_NODES, D), jnp.float32),
    ],
)


def _fuse2_body(h_ref, ws_ref, b_ref, p_ref, d_ref, o_ref):
    psum = p_ref[0] + p_ref[1]
    dsum = d_ref[0] + d_ref[1]
    rdeg = 1.0 / jnp.maximum(dsum, 1.0)
    o_ref[...] = (jnp.dot(h_ref[...], ws_ref[...], preferred_element_type=jnp.float32)
                  + psum * rdeg + b_ref[...])


_fuse2 = pl.pallas_call(
    _fuse2_body,
    grid=(N_NODES // BS,),
    in_specs=[
        pl.BlockSpec((BS, D), lambda i: (i, 0)),
        pl.BlockSpec((D, D), lambda i: (0, 0)),
        pl.BlockSpec((1, D), lambda i: (0, 0)),
        pl.BlockSpec((NC, BS, D), lambda i: (0, i, 0)),
        pl.BlockSpec((NC, BS, 1), lambda i: (0, i, 0)),
    ],
    out_specs=pl.BlockSpec((BS, D), lambda i: (i, 0)),
    out_shape=jax.ShapeDtypeStruct((N_NODES, D), jnp.float32),
)


def kernel(in_feat, edge_index, W_self1, W_neigh1, b1, W_self2, W_neigh2, b2):
    src = edge_index[0].astype(jnp.int32)
    dst = edge_index[1].astype(jnp.int32)
    pad = E_PAD - N_EDGES
    src_p = jnp.concatenate([src, jnp.zeros((pad,), jnp.int32)]).reshape(TCH, 128)
    dst_p = jnp.concatenate([dst, jnp.full((pad,), DUMMY, jnp.int32)]).reshape(TCH, 128)

    b1r = b1.reshape(1, D)
    b2r = b2.reshape(1, D)

    deg_flat = _sc_deg(dst_p)
    dgr = deg_flat.reshape(NC, ACC, D)[:, :, 0:1]

    y1 = _mm(in_feat, W_neigh1)
    p1 = _sc_agg(y1, src_p, dst_p).reshape(NC, ACC, D)
    h1, y2 = _fuse1(in_feat, W_self1, b1r, p1, dgr, W_neigh2)
    p2 = _sc_agg(y2, src_p, dst_p).reshape(NC, ACC, D)
    out = _fuse2(h1, W_self2, b2r, p2, dgr)
    return out


# speedup vs baseline: 1.1576x; 1.1576x over previous
import jax
import jax.numpy as jnp
from jax import lax
from jax.experimental import pallas as pl
from jax.experimental.pallas import tpu as pltpu, tpu_sc as plsc

N_NODES = 10000
N_EDGES = 320000
D = 128

NC = 2
NS = 16
NW = NC * NS
NCH = 80
TCH = NW * NCH
E_PAD = TCH * 128
DUMMY = N_NODES
ACC = 10240
RPT = ACC // NS

_MESH = plsc.VectorSubcoreMesh(core_axis_name="c", subcore_axis_name="s")


def _make_agg(c0, c1, qb, deg_on_c1):
    nb0 = c0 // qb
    nb1 = 0 if deg_on_c1 else c1 // qb

    def body(y_hbm, src_hbm, dst_hbm, p_out, src_v, dst_v, r0, r1, acc_s,
             s0, s1):
        cid = lax.axis_index("c")
        sid = lax.axis_index("s")
        base = sid * RPT

        zero16 = jnp.zeros((16,), jnp.float32)

        def zrow(i, _):
            def zcol(j, _):
                r0[i, pl.ds(j * 16, 16)] = zero16
                return 0
            return lax.fori_loop(0, D // 16, zcol, 0)
        lax.fori_loop(0, 128, zrow, 0)

        def zcp(k, _):
            pltpu.sync_copy(r0, acc_s.at[pl.ds(base + k * 128, 128)])
            return 0
        lax.fori_loop(0, RPT // 128, zcp, 0)

        plsc.subcore_barrier()

        def gather(sv, j, r, s):
            pltpu.async_copy(y_hbm.at[sv.at[j]], r, s)

        def gwait(r, s):
            pltpu.make_async_copy(y_hbm.at[pl.ds(0, 128)], r, s).wait()

        start = jnp.where(cid == 0, sid * c0, NS * c0 + sid * c1)
        nb = jnp.where(cid == 0, nb0, nb1)

        def block(b, _):
            cs = start + b * qb
            pltpu.sync_copy(src_hbm.at[pl.ds(cs, qb)], src_v)
            pltpu.sync_copy(dst_hbm.at[pl.ds(cs, qb)], dst_v)

            gather(src_v, 0, r0, s0)
            gather(src_v, 1, r1, s1)

            def pair(k, _):
                gwait(r0, s0)
                pltpu.sync_copy(r0, acc_s.at[dst_v.at[2 * k]], add=True)
                gather(src_v, 2 * k + 2, r0, s0)
                gwait(r1, s1)
                pltpu.sync_copy(r1, acc_s.at[dst_v.at[2 * k + 1]], add=True)
                gather(src_v, 2 * k + 3, r1, s1)
                return 0
            lax.fori_loop(0, qb // 2 - 1, pair, 0)

            gwait(r0, s0)
            pltpu.sync_copy(r0, acc_s.at[dst_v.at[qb - 2]], add=True)
            gwait(r1, s1)
            pltpu.sync_copy(r1, acc_s.at[dst_v.at[qb - 1]], add=True)
            return 0

        lax.fori_loop(0, nb, block, 0)

        if deg_on_c1:
            @pl.when(cid == 1)
            def _():
                one16 = jnp.ones((16,), jnp.float32)

                def orow(i, _):
                    def ocol(j, _):
                        r1[i, pl.ds(j * 16, 16)] = one16
                        return 0
                    return lax.fori_loop(0, D // 16, ocol, 0)
                lax.fori_loop(0, 128, orow, 0)

                dpw = TCH // NS

                def dblock(b, _):
                    pltpu.sync_copy(
                        dst_hbm.at[pl.ds(sid * dpw + b * qb, qb)], dst_v)

                    def step(j, _):
                        pltpu.sync_copy(r1, acc_s.at[dst_v.at[j]], add=True)
                        return 0
                    lax.fori_loop(0, qb, step, 0)
                    return 0
                lax.fori_loop(0, dpw // qb, dblock, 0)

        plsc.subcore_barrier()

        off = cid * ACC + base
        pltpu.sync_copy(acc_s.at[pl.ds(base, RPT)], p_out.at[pl.ds(off, RPT)])

    return pl.kernel(
        body,
        out_type=jax.ShapeDtypeStruct((NC * ACC, D), jnp.float32),
        mesh=_MESH,
        scratch_types=[
            pltpu.VMEM((qb, 128), jnp.int32),
            pltpu.VMEM((qb, 128), jnp.int32),
            pltpu.VMEM((128, D), jnp.float32),
            pltpu.VMEM((128, D), jnp.float32),
            pltpu.VMEM_SHARED((ACC, D), jnp.float32),
            pltpu.SemaphoreType.DMA,
            pltpu.SemaphoreType.DMA,
        ],
    )


_sc_agg1 = _make_agg(160, 0, 32, True)
_sc_agg2 = _make_agg(152, 8, 8, False)


BS = 2000


def _mm_body(x_ref, w_ref, o_ref):
    o_ref[...] = jnp.dot(x_ref[...], w_ref[...], preferred_element_type=jnp.float32)


_mm = pl.pallas_call(
    _mm_body,
    grid=(N_NODES // BS,),
    in_specs=[
        pl.BlockSpec((BS, D), lambda i: (i, 0)),
        pl.BlockSpec((D, D), lambda i: (0, 0)),
    ],
    out_specs=pl.BlockSpec((BS, D), lambda i: (i, 0)),
    out_shape=jax.ShapeDtypeStruct((N_NODES, D), jnp.float32),
)


def _fuse1_body(x_ref, ws_ref, b_ref, p_ref, d_ref, wn2_ref, h_ref, y2_ref):
    rdeg = 1.0 / jnp.maximum(d_ref[...], 1.0)
    h = (jnp.dot(x_ref[...], ws_ref[...], preferred_element_type=jnp.float32)
         + p_ref[...] * rdeg + b_ref[...])
    h = jnp.maximum(h, 0.0)
    h_ref[...] = h
    y2_ref[...] = jnp.dot(h, wn2_ref[...], preferred_element_type=jnp.float32)


_fuse1 = pl.pallas_call(
    _fuse1_body,
    grid=(N_NODES // BS,),
    in_specs=[
        pl.BlockSpec((BS, D), lambda i: (i, 0)),
        pl.BlockSpec((D, D), lambda i: (0, 0)),
        pl.BlockSpec((1, D), lambda i: (0, 0)),
        pl.BlockSpec((BS, D), lambda i: (i, 0)),
        pl.BlockSpec((BS, 1), lambda i: (i, 0)),
        pl.BlockSpec((D, D), lambda i: (0, 0)),
    ],
    out_specs=[
        pl.BlockSpec((BS, D), lambda i: (i, 0)),
        pl.BlockSpec((BS, D), lambda i: (i, 0)),
    ],
    out_shape=[
        jax.ShapeDtypeStruct((N_NODES, D), jnp.float32),
        jax.ShapeDtypeStruct((N_NODES, D), jnp.float32),
    ],
)


def _fuse2_body(h_ref, ws_ref, b_ref, p_ref, d_ref, o_ref):
    psum = p_ref[0] + p_ref[1]
    rdeg = 1.0 / jnp.maximum(d_ref[...], 1.0)
    o_ref[...] = (jnp.dot(h_ref[...], ws_ref[...], preferred_element_type=jnp.float32)
                  + psum * rdeg + b_ref[...])


_fuse2 = pl.pallas_call(
    _fuse2_body,
    grid=(N_NODES // BS,),
    in_specs=[
        pl.BlockSpec((BS, D), lambda i: (i, 0)),
        pl.BlockSpec((D, D), lambda i: (0, 0)),
        pl.BlockSpec((1, D), lambda i: (0, 0)),
        pl.BlockSpec((NC, BS, D), lambda i: (0, i, 0)),
        pl.BlockSpec((BS, 1), lambda i: (i, 0)),
    ],
    out_specs=pl.BlockSpec((BS, D), lambda i: (i, 0)),
    out_shape=jax.ShapeDtypeStruct((N_NODES, D), jnp.float32),
)


def kernel(in_feat, edge_index, W_self1, W_neigh1, b1, W_self2, W_neigh2, b2):
    src = edge_index[0].astype(jnp.int32)
    dst = edge_index[1].astype(jnp.int32)
    pad = E_PAD - N_EDGES
    src_p = jnp.concatenate([src, jnp.zeros((pad,), jnp.int32)]).reshape(TCH, 128)
    dst_p = jnp.concatenate([dst, jnp.full((pad,), DUMMY, jnp.int32)]).reshape(TCH, 128)

    b1r = b1.reshape(1, D)
    b2r = b2.reshape(1, D)

    y1 = _mm(in_feat, W_neigh1)
    pd = _sc_agg1(y1, src_p, dst_p).reshape(NC, ACC, D)
    p1 = pd[0]
    dgr = pd[1, :, 0:1]
    h1, y2 = _fuse1(in_feat, W_self1, b1r, p1, dgr, W_neigh2)
    p2 = _sc_agg2(y2, src_p, dst_p).reshape(NC, ACC, D)
    out = _fuse2(h1, W_self2, b2r, p2, dgr)
    return out


# speedup vs baseline: 1.3179x; 1.1384x over previous
import jax
import jax.numpy as jnp
from jax import lax
from jax.experimental import pallas as pl
from jax.experimental.pallas import tpu as pltpu, tpu_sc as plsc

N_NODES = 10000
N_EDGES = 320000
D = 128

NC = 2
NS = 16
NW = NC * NS
NCH = 80
TCH = NW * NCH
E_PAD = TCH * 128
DUMMY = N_NODES
ACC = 10240
RPT = ACC // NS

_MESH = plsc.VectorSubcoreMesh(core_axis_name="c", subcore_axis_name="s")


def _make_agg(c0, c1, qb, deg_on_c1):
    nb0 = c0 // qb
    nb1 = 0 if deg_on_c1 else c1 // qb

    def body(y_hbm, src_hbm, dst_hbm, p_out, src_v, dst_v, r0, r1, acc_s,
             s0, s1):
        cid = lax.axis_index("c")
        sid = lax.axis_index("s")
        base = sid * RPT

        zero16 = jnp.zeros((16,), jnp.float32)

        def zrow(i, _):
            def zcol(j, _):
                r0[i, pl.ds(j * 16, 16)] = zero16
                return 0
            return lax.fori_loop(0, D // 16, zcol, 0)
        lax.fori_loop(0, 128, zrow, 0)

        def zcp(k, _):
            pltpu.sync_copy(r0, acc_s.at[pl.ds(base + k * 128, 128)])
            return 0
        lax.fori_loop(0, RPT // 128, zcp, 0)

        plsc.subcore_barrier()

        def gather(sv, j, r, s):
            pltpu.async_copy(y_hbm.at[sv.at[j]], r, s)

        def gwait(r, s):
            pltpu.make_async_copy(y_hbm.at[pl.ds(0, 128)], r, s).wait()

        start = jnp.where(cid == 0, sid * c0, NS * c0 + sid * c1)
        nb = jnp.where(cid == 0, nb0, nb1)

        def block(b, _):
            cs = start + b * qb
            pltpu.sync_copy(src_hbm.at[pl.ds(cs, qb)], src_v)
            pltpu.sync_copy(dst_hbm.at[pl.ds(cs, qb)], dst_v)

            gather(src_v, 0, r0, s0)
            gather(src_v, 1, r1, s1)

            def pair(k, _):
                gwait(r0, s0)
                pltpu.sync_copy(r0, acc_s.at[dst_v.at[2 * k]], add=True)
                gather(src_v, 2 * k + 2, r0, s0)
                gwait(r1, s1)
                pltpu.sync_copy(r1, acc_s.at[dst_v.at[2 * k + 1]], add=True)
                gather(src_v, 2 * k + 3, r1, s1)
                return 0
            lax.fori_loop(0, qb // 2 - 1, pair, 0)

            gwait(r0, s0)
            pltpu.sync_copy(r0, acc_s.at[dst_v.at[qb - 2]], add=True)
            gwait(r1, s1)
            pltpu.sync_copy(r1, acc_s.at[dst_v.at[qb - 1]], add=True)
            return 0

        lax.fori_loop(0, nb, block, 0)

        if deg_on_c1:
            @pl.when(cid == 1)
            def _():
                one16 = jnp.ones((16,), jnp.float32)

                def orow(i, _):
                    def ocol(j, _):
                        r1[i, pl.ds(j * 16, 16)] = one16
                        return 0
                    return lax.fori_loop(0, D // 16, ocol, 0)
                lax.fori_loop(0, 128, orow, 0)

                dpw = TCH // NS

                def dblock(b, _):
                    pltpu.sync_copy(
                        dst_hbm.at[pl.ds(sid * dpw + b * qb, qb)], dst_v)

                    def step(j, _):
                        pltpu.sync_copy(r1, acc_s.at[dst_v.at[j]], add=True)
                        return 0
                    lax.fori_loop(0, qb, step, 0)
                    return 0
                lax.fori_loop(0, dpw // qb, dblock, 0)

        plsc.subcore_barrier()

        off = cid * ACC + base
        pltpu.sync_copy(acc_s.at[pl.ds(base, RPT)], p_out.at[pl.ds(off, RPT)])

    return pl.kernel(
        body,
        out_type=jax.ShapeDtypeStruct((NC * ACC, D), jnp.float32),
        mesh=_MESH,
        scratch_types=[
            pltpu.VMEM((qb, 128), jnp.int32),
            pltpu.VMEM((qb, 128), jnp.int32),
            pltpu.VMEM((128, D), jnp.float32),
            pltpu.VMEM((128, D), jnp.float32),
            pltpu.VMEM_SHARED((ACC, D), jnp.float32),
            pltpu.SemaphoreType.DMA,
            pltpu.SemaphoreType.DMA,
        ],
    )


_sc_agg = _make_agg(152, 8, 8, False)


def _deg_body(dst_hbm, d_out, dst_v, ones_v, zbuf, dacc_s):
    cid = lax.axis_index("c")
    sid = lax.axis_index("s")
    wid = cid * NS + sid
    base = sid * RPT

    zero16 = jnp.zeros((16,), jnp.float32)
    one16 = jnp.ones((16,), jnp.float32)

    def zrow(i, _):
        def zcol(j, _):
            zbuf[i, pl.ds(j * 16, 16)] = zero16
            return 0
        return lax.fori_loop(0, D // 16, zcol, 0)
    lax.fori_loop(0, 16, zrow, 0)

    def orow(i, _):
        def ocol(j, _):
            ones_v[i, pl.ds(j * 16, 16)] = one16
            return 0
        return lax.fori_loop(0, D // 16, ocol, 0)
    lax.fori_loop(0, 128, orow, 0)

    def zcd(k, _):
        pltpu.sync_copy(zbuf, dacc_s.at[pl.ds(base + k * 16, 16)])
        return 0
    lax.fori_loop(0, RPT // 16, zcd, 0)

    plsc.subcore_barrier()

    def quarter(q):
        pltpu.sync_copy(dst_hbm.at[pl.ds(wid * NCH + q * 16, 16)], dst_v)

        def step(j, _):
            pltpu.sync_copy(ones_v, dacc_s.at[dst_v.at[j]], add=True)
            return 0
        lax.fori_loop(0, 16, step, 0)

    for q in range(5):
        quarter(q)

    plsc.subcore_barrier()

    off = cid * ACC + base
    pltpu.sync_copy(dacc_s.at[pl.ds(base, RPT)], d_out.at[pl.ds(off, RPT)])


_sc_deg = pl.kernel(
    _deg_body,
    out_type=jax.ShapeDtypeStruct((NC * ACC, D), jnp.float32),
    mesh=_MESH,
    scratch_types=[
        pltpu.VMEM((16, 128), jnp.int32),
        pltpu.VMEM((128, D), jnp.float32),
        pltpu.VMEM((16, D), jnp.float32),
        pltpu.VMEM_SHARED((ACC, D), jnp.float32),
    ],
)


BS = 2000


def _mm_body(x_ref, w_ref, o_ref):
    o_ref[...] = jnp.dot(x_ref[...], w_ref[...], preferred_element_type=jnp.float32)


_mm = pl.pallas_call(
    _mm_body,
    grid=(N_NODES // BS,),
    in_specs=[
        pl.BlockSpec((BS, D), lambda i: (i, 0)),
        pl.BlockSpec((D, D), lambda i: (0, 0)),
    ],
    out_specs=pl.BlockSpec((BS, D), lambda i: (i, 0)),
    out_shape=jax.ShapeDtypeStruct((N_NODES, D), jnp.float32),
)


def _fuse1_body(x_ref, ws_ref, b_ref, p_ref, d_ref, wn2_ref, h_ref, y2_ref):
    psum = p_ref[0] + p_ref[1]
    rdeg = 1.0 / jnp.maximum(d_ref[0] + d_ref[1], 1.0)
    h = (jnp.dot(x_ref[...], ws_ref[...], preferred_element_type=jnp.float32)
         + psum * rdeg + b_ref[...])
    h = jnp.maximum(h, 0.0)
    h_ref[...] = h
    y2_ref[...] = jnp.dot(h, wn2_ref[...], preferred_element_type=jnp.float32)


_fuse1 = pl.pallas_call(
    _fuse1_body,
    grid=(N_NODES // BS,),
    in_specs=[
        pl.BlockSpec((BS, D), lambda i: (i, 0)),
        pl.BlockSpec((D, D), lambda i: (0, 0)),
        pl.BlockSpec((1, D), lambda i: (0, 0)),
        pl.BlockSpec((NC, BS, D), lambda i: (0, i, 0)),
        pl.BlockSpec((NC, BS, 1), lambda i: (0, i, 0)),
        pl.BlockSpec((D, D), lambda i: (0, 0)),
    ],
    out_specs=[
        pl.BlockSpec((BS, D), lambda i: (i, 0)),
        pl.BlockSpec((BS, D), lambda i: (i, 0)),
    ],
    out_shape=[
        jax.ShapeDtypeStruct((N_NODES, D), jnp.float32),
        jax.ShapeDtypeStruct((N_NODES, D), jnp.float32),
    ],
)


def _fuse2_body(h_ref, ws_ref, b_ref, p_ref, d_ref, o_ref):
    psum = p_ref[0] + p_ref[1]
    rdeg = 1.0 / jnp.maximum(d_ref[0] + d_ref[1], 1.0)
    o_ref[...] = (jnp.dot(h_ref[...], ws_ref[...], preferred_element_type=jnp.float32)
                  + psum * rdeg + b_ref[...])


_fuse2 = pl.pallas_call(
    _fuse2_body,
    grid=(N_NODES // BS,),
    in_specs=[
        pl.BlockSpec((BS, D), lambda i: (i, 0)),
        pl.BlockSpec((D, D), lambda i: (0, 0)),
        pl.BlockSpec((1, D), lambda i: (0, 0)),
        pl.BlockSpec((NC, BS, D), lambda i: (0, i, 0)),
        pl.BlockSpec((NC, BS, 1), lambda i: (0, i, 0)),
    ],
    out_specs=pl.BlockSpec((BS, D), lambda i: (i, 0)),
    out_shape=jax.ShapeDtypeStruct((N_NODES, D), jnp.float32),
)


def kernel(in_feat, edge_index, W_self1, W_neigh1, b1, W_self2, W_neigh2, b2):
    src = edge_index[0].astype(jnp.int32)
    dst = edge_index[1].astype(jnp.int32)
    pad = E_PAD - N_EDGES
    src_p = jnp.concatenate([src, jnp.zeros((pad,), jnp.int32)]).reshape(TCH, 128)
    dst_p = jnp.concatenate([dst, jnp.full((pad,), DUMMY, jnp.int32)]).reshape(TCH, 128)

    b1r = b1.reshape(1, D)
    b2r = b2.reshape(1, D)

    dgr = _sc_deg(dst_p).reshape(NC, ACC, D)[:, :, 0:1]
    y1 = _mm(in_feat, W_neigh1)
    p1 = _sc_agg(y1, src_p, dst_p).reshape(NC, ACC, D)
    h1, y2 = _fuse1(in_feat, W_self1, b1r, p1, dgr, W_neigh2)
    p2 = _sc_agg(y2, src_p, dst_p).reshape(NC, ACC, D)
    out = _fuse2(h1, W_self2, b2r, p2, dgr)
    return out
